# baseline (speedup 1.0000x reference)
import jax
import jax.numpy as jnp
from jax import lax
from jax.experimental import pallas as pl
from jax.experimental.pallas import tpu as pltpu, tpu_sc as plsc

N = 10000
E = 320000
D = 128
NPAD = 10240
NC = 2
NS = 16
NW = NC * NS
EPW = E // NW
CB = 40
NCHUNK = EPW // CB
NBLK = 5
BCH = NCHUNK // NBLK
RPT = NPAD // NS


def _sc_body(x_hbm, cat_hbm, src_hbm, acc_out, cnt_out,
             idx_c, idx_s, g0, g1, m0, czero, ones_v,
             sem_g0, sem_g1, acc_sh, cnt_sh):
    c = lax.axis_index("c")
    s = lax.axis_index("s")
    wid = c * NS + s

    def _zrow(r, _):
        for j in range(D // 16):
            m0[r, pl.ds(j * 16, 16)] = jnp.zeros((16,), jnp.float32)
        return 0
    lax.fori_loop(0, CB, _zrow, 0)

    def _zc(r, _):
        czero[pl.ds(r * 16, 16)] = jnp.zeros((16,), jnp.float32)
        return 0
    lax.fori_loop(0, RPT // 16, _zc, 0)

    for j in range(3):
        ones_v[pl.ds(j * 16, 16)] = jnp.ones((16,), jnp.float32)

    base = s * RPT
    for t in range(RPT // CB):
        pltpu.sync_copy(m0, acc_sh.at[pl.ds(base + t * CB, CB)])
    pltpu.sync_copy(czero, cnt_sh.at[pl.ds(base, RPT)])
    plsc.subcore_barrier()

    def _wait(buf, sem):
        pltpu.make_async_copy(x_hbm.at[pl.ds(0, 2 * CB)], buf, sem).wait()

    def _compute(gb):
        def _row(r, _):
            for j in range(D // 16):
                sl = pl.ds(j * 16, 16)
                m0[r, sl] = jnp.abs(gb[r, sl] - gb[r + CB, sl])
            return 0
        lax.fori_loop(0, CB, _row, 0)

    def _chunk(t, gb, sem, gnext, semnext, last):
        _wait(gb, sem)

        @pl.when(jnp.logical_not(last))
        def _():
            pltpu.async_copy(x_hbm.at[idx_c.at[t + 1]], gnext, semnext)

        _compute(gb)
        pltpu.sync_copy(m0, acc_sh.at[idx_s.at[t]], add=True)
        pltpu.sync_copy(ones_v.at[pl.ds(0, CB)],
                        cnt_sh.at[idx_s.at[t]], add=True)

    def _block(blk, _):
        pltpu.sync_copy(cat_hbm.at[wid, blk], idx_c)
        pltpu.sync_copy(src_hbm.at[wid, blk], idx_s)
        pltpu.async_copy(x_hbm.at[idx_c.at[0]], g0, sem_g0)

        def _pair(k2, __):
            e = 2 * k2
            _chunk(e, g0, sem_g0, g1, sem_g1, jnp.bool_(False))
            _chunk(e + 1, g1, sem_g1, g0, sem_g0, k2 >= BCH // 2 - 1)
            return 0
        lax.fori_loop(0, BCH // 2, _pair, 0)
        return 0
    lax.fori_loop(0, NBLK, _block, 0)

    plsc.subcore_barrier()
    pltpu.sync_copy(acc_sh.at[pl.ds(base, RPT)],
                    acc_out.at[c].at[pl.ds(base, RPT)])
    pltpu.sync_copy(cnt_sh.at[pl.ds(base, RPT)],
                    cnt_out.at[pl.ds(c * NPAD + base, RPT)])


_sc_gather_scatter = pl.kernel(
    _sc_body,
    out_type=(
        jax.ShapeDtypeStruct((NC, NPAD, D), jnp.float32),
        jax.ShapeDtypeStruct((NC * NPAD,), jnp.float32),
    ),
    mesh=plsc.VectorSubcoreMesh(core_axis_name="c", subcore_axis_name="s"),
    scratch_types=[
        pltpu.VMEM((BCH, 2 * CB), jnp.int32),
        pltpu.VMEM((BCH, CB), jnp.int32),
        pltpu.VMEM((2 * CB, D), jnp.float32),
        pltpu.VMEM((2 * CB, D), jnp.float32),
        pltpu.VMEM((CB, D), jnp.float32),
        pltpu.VMEM((RPT,), jnp.float32),
        pltpu.VMEM((48,), jnp.float32),
        pltpu.SemaphoreType.DMA,
        pltpu.SemaphoreType.DMA,
        pltpu.VMEM_SHARED((NPAD, D), jnp.float32),
        pltpu.VMEM_SHARED((NPAD,), jnp.float32),
    ],
)


BLK = 512


def _tc_body(x_ref, acc_ref, cnt_ref, w_ref, b_ref, o_ref):
    a = x_ref[...] + acc_ref[0] + acc_ref[1]
    denom = cnt_ref[...].sum(axis=1, keepdims=True) + 1.0
    m = a / denom
    o_ref[...] = lax.dot_general(
        m, w_ref[...], (((1,), (1,)), ((), ())),
        preferred_element_type=jnp.float32) + b_ref[...]


_tc_combine = pl.pallas_call(
    _tc_body,
    grid=(NPAD // BLK,),
    in_specs=[
        pl.BlockSpec((BLK, D), lambda i: (i, 0)),
        pl.BlockSpec((NC, BLK, D), lambda i: (0, i, 0)),
        pl.BlockSpec((BLK, NC), lambda i: (i, 0)),
        pl.BlockSpec((D, D), lambda i: (0, 0)),
        pl.BlockSpec((1, D), lambda i: (0, 0)),
    ],
    out_specs=pl.BlockSpec((BLK, D), lambda i: (i, 0)),
    out_shape=jax.ShapeDtypeStruct((NPAD, D), jnp.float32),
    compiler_params=pltpu.CompilerParams(
        dimension_semantics=("arbitrary",)),
)


def kernel(x, edge_index, W, b):
    src = edge_index[0].reshape(NW, NBLK, BCH, CB)
    dst = edge_index[1].reshape(NW, NBLK, BCH, CB)
    cat = jnp.concatenate([src, dst], axis=-1)
    acc, cnt = _sc_gather_scatter(x, cat, src)
    xp = jnp.pad(x, ((0, NPAD - N), (0, 0)))
    out = _tc_combine(xp, acc, cnt.reshape(NC, NPAD).T, W, b.reshape(1, D))
    return out[:N]


# speedup vs baseline: 1.2777x; 1.2777x over previous
import jax
import jax.numpy as jnp
from jax import lax
from jax.experimental import pallas as pl
from jax.experimental.pallas import tpu as pltpu, tpu_sc as plsc

N = 10000
E = 320000
D = 128
NPAD = 10240
NC = 2
NS = 16
NW = NC * NS
EPW = E // NW
CB = 16
NCHUNK = EPW // CB
NBLK = 25
BCH = NCHUNK // NBLK
DEPTH = 5
RPT = NPAD // NS


def _sc_body(x_hbm, src_hbm, dst_hbm, acc_out, cnt_out,
             idx_s, idx_d, s0, s1, s2, s3, s4, d0, d1, d2, d3, d4, m0, m1,
             czero, ones_v,
             sem_s0, sem_s1, sem_s2, sem_s3, sem_s4,
             sem_d0, sem_d1, sem_d2, sem_d3, sem_d4,
             sem_m0, sem_m1, sem_c0, sem_c1,
             acc_sh, cnt_sh):
    c = lax.axis_index("c")
    s = lax.axis_index("s")
    wid = c * NS + s
    sbufs = (s0, s1, s2, s3, s4)
    dbufs = (d0, d1, d2, d3, d4)
    ssems = (sem_s0, sem_s1, sem_s2, sem_s3, sem_s4)
    dsems = (sem_d0, sem_d1, sem_d2, sem_d3, sem_d4)
    mbufs = (m0, m1)
    msems = (sem_m0, sem_m1)
    csems = (sem_c0, sem_c1)

    def _zrow(r, _):
        for j in range(D // 16):
            m0[r, pl.ds(j * 16, 16)] = jnp.zeros((16,), jnp.float32)
        return 0
    lax.fori_loop(0, CB, _zrow, 0)

    def _zc(r, _):
        czero[pl.ds(r * 16, 16)] = jnp.zeros((16,), jnp.float32)
        return 0
    lax.fori_loop(0, RPT // 16, _zc, 0)

    ones_v[pl.ds(0, 16)] = jnp.ones((16,), jnp.float32)

    base = s * RPT
    for t in range(RPT // CB):
        pltpu.sync_copy(m0, acc_sh.at[pl.ds(base + t * CB, CB)])
    pltpu.sync_copy(czero, cnt_sh.at[pl.ds(base, RPT)])
    plsc.subcore_barrier()

    def _wait(buf, sem):
        pltpu.make_async_copy(x_hbm.at[pl.ds(0, CB)], buf, sem).wait()

    def _drain_m(p):
        pltpu.make_async_copy(mbufs[p], acc_sh.at[idx_s.at[0]],
                              msems[p]).wait()
        pltpu.make_async_copy(ones_v, cnt_sh.at[idx_s.at[0]],
                              csems[p]).wait()

    def _compute(sb, db, mb):
        def _row(r, _):
            for j in range(D // 16):
                sl = pl.ds(j * 16, 16)
                mb[r, sl] = jnp.abs(sb[r, sl] - db[r, sl])
            return 0
        lax.fori_loop(0, CB, _row, 0)

    def _block(blk, _):
        pltpu.sync_copy(src_hbm.at[wid, blk], idx_s)
        pltpu.sync_copy(dst_hbm.at[wid, blk], idx_d)
        for i in range(DEPTH):
            pltpu.async_copy(x_hbm.at[idx_s.at[i]], sbufs[i], ssems[i])
            pltpu.async_copy(x_hbm.at[idx_d.at[i]], dbufs[i], dsems[i])

        def _quint(k5, __):
            for i in range(DEPTH):
                t = DEPTH * k5 + i
                p = i % 2
                _wait(sbufs[i], ssems[i])
                _wait(dbufs[i], dsems[i])
                if i < 2:
                    @pl.when(k5 > 0)
                    def _():
                        _drain_m(p)
                else:
                    _drain_m(p)
                _compute(sbufs[i], dbufs[i], mbufs[p])
                pltpu.async_copy(mbufs[p], acc_sh.at[idx_s.at[t]],
                                 msems[p], add=True)
                pltpu.async_copy(ones_v, cnt_sh.at[idx_s.at[t]],
                                 csems[p], add=True)

                @pl.when(k5 < BCH // DEPTH - 1)
                def _():
                    pltpu.async_copy(x_hbm.at[idx_s.at[t + DEPTH]],
                                     sbufs[i], ssems[i])
                    pltpu.async_copy(x_hbm.at[idx_d.at[t + DEPTH]],
                                     dbufs[i], dsems[i])
            return 0
        lax.fori_loop(0, BCH // DEPTH, _quint, 0)
        _drain_m(0)
        _drain_m(1)
        return 0
    lax.fori_loop(0, NBLK, _block, 0)

    plsc.subcore_barrier()
    pltpu.sync_copy(acc_sh.at[pl.ds(base, RPT)],
                    acc_out.at[c].at[pl.ds(base, RPT)])
    pltpu.sync_copy(cnt_sh.at[pl.ds(base, RPT)],
                    cnt_out.at[pl.ds(c * NPAD + base, RPT)])


_sc_gather_scatter = pl.kernel(
    _sc_body,
    out_type=(
        jax.ShapeDtypeStruct((NC, NPAD, D), jnp.float32),
        jax.ShapeDtypeStruct((NC * NPAD,), jnp.float32),
    ),
    mesh=plsc.VectorSubcoreMesh(core_axis_name="c", subcore_axis_name="s"),
    scratch_types=(
        [pltpu.VMEM((BCH, CB), jnp.int32)] * 2
        + [pltpu.VMEM((CB, D), jnp.float32)] * 12
        + [pltpu.VMEM((RPT,), jnp.float32), pltpu.VMEM((16,), jnp.float32)]
        + [pltpu.SemaphoreType.DMA] * 14
        + [pltpu.VMEM_SHARED((NPAD, D), jnp.float32),
           pltpu.VMEM_SHARED((NPAD,), jnp.float32)]
    ),
)


BLK = 512


def _tc_body(x_ref, acc_ref, cnt_ref, w_ref, b_ref, o_ref):
    a = x_ref[...] + acc_ref[0] + acc_ref[1]
    denom = cnt_ref[...].sum(axis=1, keepdims=True) + 1.0
    m = a / denom
    o_ref[...] = lax.dot_general(
        m, w_ref[...], (((1,), (1,)), ((), ())),
        preferred_element_type=jnp.float32) + b_ref[...]


_tc_combine = pl.pallas_call(
    _tc_body,
    grid=(NPAD // BLK,),
    in_specs=[
        pl.BlockSpec((BLK, D), lambda i: (i, 0)),
        pl.BlockSpec((NC, BLK, D), lambda i: (0, i, 0)),
        pl.BlockSpec((BLK, NC), lambda i: (i, 0)),
        pl.BlockSpec((D, D), lambda i: (0, 0)),
        pl.BlockSpec((1, D), lambda i: (0, 0)),
    ],
    out_specs=pl.BlockSpec((BLK, D), lambda i: (i, 0)),
    out_shape=jax.ShapeDtypeStruct((NPAD, D), jnp.float32),
    compiler_params=pltpu.CompilerParams(
        dimension_semantics=("arbitrary",)),
)


def kernel(x, edge_index, W, b):
    src = edge_index[0].reshape(NW, NBLK, BCH, CB)
    dst = edge_index[1].reshape(NW, NBLK, BCH, CB)
    acc, cnt = _sc_gather_scatter(x, src, dst)
    xp = jnp.pad(x, ((0, NPAD - N), (0, 0)))
    out = _tc_combine(xp, acc, cnt.reshape(NC, NPAD).T, W, b.reshape(1, D))
    return out[:N]


# speedup vs baseline: 1.2972x; 1.0153x over previous
import jax
import jax.numpy as jnp
from jax import lax
from jax.experimental import pallas as pl
from jax.experimental.pallas import tpu as pltpu, tpu_sc as plsc

N = 10000
E = 320000
D = 128
NPAD = 10240
NC = 2
NS = 16
NW = NC * NS
EPW = E // NW
CB = 16
NCHUNK = EPW // CB
NBLK = 25
BCH = NCHUNK // NBLK
DEPTH = 5
RPT = NPAD // NS


def _sc_body(x_hbm, src_hbm, dst_hbm, acc_out, cnt_out,
             idx_s, idx_d, s0, s1, s2, s3, s4, d0, d1, d2, d3, d4, m0, m1,
             czero, ones_v,
             sem_s0, sem_s1, sem_s2, sem_s3, sem_s4,
             sem_d0, sem_d1, sem_d2, sem_d3, sem_d4,
             sem_m0, sem_m1, sem_c0, sem_c1,
             acc_sh, cnt_sh):
    c = lax.axis_index("c")
    s = lax.axis_index("s")
    wid = c * NS + s
    sbufs = (s0, s1, s2, s3, s4)
    dbufs = (d0, d1, d2, d3, d4)
    ssems = (sem_s0, sem_s1, sem_s2, sem_s3, sem_s4)
    dsems = (sem_d0, sem_d1, sem_d2, sem_d3, sem_d4)
    mbufs = (m0, m1)
    msems = (sem_m0, sem_m1)
    csems = (sem_c0, sem_c1)

    def _zrow(r, _):
        for j in range(D // 16):
            m0[r, pl.ds(j * 16, 16)] = jnp.zeros((16,), jnp.float32)
        return 0
    lax.fori_loop(0, CB, _zrow, 0)

    def _zc(r, _):
        czero[pl.ds(r * 16, 16)] = jnp.zeros((16,), jnp.float32)
        return 0
    lax.fori_loop(0, RPT // 16, _zc, 0)

    ones_v[pl.ds(0, 16)] = jnp.ones((16,), jnp.float32)

    base = s * RPT
    for t in range(RPT // CB):
        pltpu.sync_copy(m0, acc_sh.at[pl.ds(base + t * CB, CB)])
    pltpu.sync_copy(czero, cnt_sh.at[pl.ds(base, RPT)])
    plsc.subcore_barrier()

    def _wait(buf, sem):
        pltpu.make_async_copy(x_hbm.at[pl.ds(0, CB)], buf, sem).wait()

    def _drain_m(p):
        pltpu.make_async_copy(mbufs[p], acc_sh.at[idx_s.at[0]],
                              msems[p]).wait()
        pltpu.make_async_copy(ones_v, cnt_sh.at[idx_s.at[0]],
                              csems[p]).wait()

    def _compute(sb, db, mb):
        def _row(r, _):
            for j in range(D // 16):
                sl = pl.ds(j * 16, 16)
                mb[r, sl] = jnp.abs(sb[r, sl] - db[r, sl])
            return 0
        lax.fori_loop(0, CB, _row, 0)

    def _block(blk, _):
        pltpu.sync_copy(src_hbm.at[wid, blk], idx_s)
        pltpu.sync_copy(dst_hbm.at[wid, blk], idx_d)
        for i in range(DEPTH):
            pltpu.async_copy(x_hbm.at[idx_s.at[i]], sbufs[i], ssems[i])
            pltpu.async_copy(x_hbm.at[idx_d.at[i]], dbufs[i], dsems[i])

        def _quint(k5, __):
            for i in range(DEPTH):
                t = DEPTH * k5 + i
                p = i % 2
                _wait(sbufs[i], ssems[i])
                _wait(dbufs[i], dsems[i])
                if i < 2:
                    @pl.when(k5 > 0)
                    def _():
                        _drain_m(p)
                else:
                    _drain_m(p)
                _compute(sbufs[i], dbufs[i], mbufs[p])
                pltpu.async_copy(mbufs[p], acc_sh.at[idx_s.at[t]],
                                 msems[p], add=True)
                pltpu.async_copy(ones_v, cnt_sh.at[idx_s.at[t]],
                                 csems[p], add=True)

                @pl.when(k5 < BCH // DEPTH - 1)
                def _():
                    pltpu.async_copy(x_hbm.at[idx_s.at[t + DEPTH]],
                                     sbufs[i], ssems[i])
                    pltpu.async_copy(x_hbm.at[idx_d.at[t + DEPTH]],
                                     dbufs[i], dsems[i])
            return 0
        lax.fori_loop(0, BCH // DEPTH, _quint, 0)
        _drain_m(0)
        _drain_m(1)
        return 0
    lax.fori_loop(0, NBLK, _block, 0)

    plsc.subcore_barrier()
    pltpu.sync_copy(acc_sh.at[pl.ds(base, RPT)],
                    acc_out.at[c].at[pl.ds(base, RPT)])
    pltpu.sync_copy(cnt_sh.at[pl.ds(base, RPT)],
                    cnt_out.at[pl.ds(c * NPAD + base, RPT)])


_sc_gather_scatter = pl.kernel(
    _sc_body,
    out_type=(
        jax.ShapeDtypeStruct((NC, NPAD, D), jnp.float32),
        jax.ShapeDtypeStruct((NC * NPAD,), jnp.float32),
    ),
    mesh=plsc.VectorSubcoreMesh(core_axis_name="c", subcore_axis_name="s"),
    scratch_types=(
        [pltpu.VMEM((BCH, CB), jnp.int32)] * 2
        + [pltpu.VMEM((CB, D), jnp.float32)] * 12
        + [pltpu.VMEM((RPT,), jnp.float32), pltpu.VMEM((16,), jnp.float32)]
        + [pltpu.SemaphoreType.DMA] * 14
        + [pltpu.VMEM_SHARED((NPAD, D), jnp.float32),
           pltpu.VMEM_SHARED((NPAD,), jnp.float32)]
    ),
)


BLK = 400


def _tc_body(x_ref, acc_ref, cnt_ref, w_ref, b_ref, o_ref):
    a = x_ref[...] + acc_ref[0] + acc_ref[1]
    denom = cnt_ref[...].sum(axis=1, keepdims=True) + 1.0
    m = a / denom
    o_ref[...] = lax.dot_general(
        m, w_ref[...], (((1,), (1,)), ((), ())),
        preferred_element_type=jnp.float32) + b_ref[...]


_tc_combine = pl.pallas_call(
    _tc_body,
    grid=(N // BLK,),
    in_specs=[
        pl.BlockSpec((BLK, D), lambda i: (i, 0)),
        pl.BlockSpec((NC, BLK, D), lambda i: (0, i, 0)),
        pl.BlockSpec((BLK, NC), lambda i: (i, 0)),
        pl.BlockSpec((D, D), lambda i: (0, 0)),
        pl.BlockSpec((1, D), lambda i: (0, 0)),
    ],
    out_specs=pl.BlockSpec((BLK, D), lambda i: (i, 0)),
    out_shape=jax.ShapeDtypeStruct((N, D), jnp.float32),
    compiler_params=pltpu.CompilerParams(
        dimension_semantics=("arbitrary",)),
)


def kernel(x, edge_index, W, b):
    src = edge_index[0].reshape(NW, NBLK, BCH, CB)
    dst = edge_index[1].reshape(NW, NBLK, BCH, CB)
    acc, cnt = _sc_gather_scatter(x, src, dst)
    return _tc_combine(x, acc, cnt.reshape(NC, NPAD).T, W, b.reshape(1, D))
